# Initial kernel scaffold; baseline (speedup 1.0000x reference)
import functools

import jax
import jax.numpy as jnp
from jax import lax
from jax.experimental import pallas as pl
from jax.experimental.pallas import tpu as pltpu
from jax.experimental.pallas import tpu_sc as plsc

_N = 50000
_E = 800000
_NG = 512
_H = 64

_NCORES = 2
_NTILES = 16
_CH = 80
_SUB = 25
_EPT = _E // _NTILES
_CHUNKS = _EPT // _CH
_SUPS = _CHUNKS // _SUB
_STRIPE = _N // _NTILES
_ZR = 125


def _make_msg_kernel(dh):
    nreg = dh // 16
    mesh = plsc.VectorSubcoreMesh(core_axis_name="c", subcore_axis_name="s")

    @functools.partial(
        pl.kernel,
        out_type=jax.ShapeDtypeStruct((_NCORES, _N, dh), jnp.float32),
        mesh=mesh,
        scratch_types=[
            pltpu.VMEM((_SUB, _CH), jnp.int32),
            pltpu.VMEM((_SUB, _CH), jnp.int32),
            pltpu.VMEM((_SUB, _CH), jnp.float32),
            pltpu.VMEM((_CH, dh), jnp.float32),
            pltpu.VMEM((2, dh), jnp.float32),
            pltpu.VMEM((_ZR, dh), jnp.float32),
            pltpu.VMEM_SHARED((_N, dh), jnp.float32),
            pltpu.SemaphoreType.DMA,
        ],
    )
    def msg(tbl, srcm, dstm, am, wb, out, srcb, dstb, ab, rows, wbv, zb, agg,
            gsem):
        c = lax.axis_index("c")
        t = lax.axis_index("s")
        zi = jnp.zeros((16,), jnp.int32)
        zf = jnp.zeros((16,), jnp.float32)

        pltpu.sync_copy(wb.at[c], wbv)

        def zrow(i, carry):
            for r in range(nreg):
                zb[i, pl.ds(r * 16, 16)] = zf
            return carry
        lax.fori_loop(0, _ZR, zrow, 0)

        def zcopy(j, carry):
            pltpu.sync_copy(zb, agg.at[pl.ds(t * _STRIPE + j * _ZR, _ZR)])
            return carry
        lax.fori_loop(0, _STRIPE // _ZR, zcopy, 0)
        plsc.subcore_barrier()

        wregs = [wbv[0, pl.ds(r * 16, 16)] for r in range(nreg)]
        bregs = [wbv[1, pl.ds(r * 16, 16)] for r in range(nreg)]

        def super_body(s, carry):
            r0 = t * _CHUNKS + s * _SUB
            pltpu.sync_copy(srcm.at[pl.ds(r0, _SUB)], srcb)
            pltpu.sync_copy(dstm.at[pl.ds(r0, _SUB)], dstb)
            pltpu.sync_copy(am.at[pl.ds(r0, _SUB)], ab)
            for kk in range(_SUB):
                pltpu.async_copy(tbl.at[c].at[srcb.at[kk]], rows, gsem).wait()

                def edge_body(e, cy):
                    av = plsc.load_gather(ab, [zi + kk, zi + e])
                    for r in range(nreg):
                        xv = rows[e, pl.ds(r * 16, 16)]
                        rows[e, pl.ds(r * 16, 16)] = jnp.maximum(
                            xv + av * wregs[r] + bregs[r], 0.0)
                    return cy
                lax.fori_loop(0, _CH, edge_body, 0)
                pltpu.sync_copy(rows, agg.at[dstb.at[kk]], add=True)
            return carry
        lax.fori_loop(0, _SUPS, super_body, 0)

        plsc.subcore_barrier()
        pltpu.sync_copy(agg.at[pl.ds(t * _STRIPE, _STRIPE)],
                        out.at[c].at[pl.ds(t * _STRIPE, _STRIPE)])

    return msg


_BR = 400
_NB = _N // _BR


def _mlp_mid(xres, agg, w1, b1, w2, b2, scale, beta, eps, din, dh_pad):
    first = din != _H

    def body(x_ref, a_ref, w1_ref, b1_ref, w2_ref, b2_ref, s_ref, be_ref,
             e_ref, o_ref):
        if first:
            xb = x_ref[...]
            ab = jnp.concatenate(
                [a_ref[0, :, : din // 2], a_ref[1, :, : din // 2]], axis=1)
        else:
            xb = jnp.concatenate([x_ref[0], x_ref[1]], axis=1)
            ab = jnp.concatenate([a_ref[0], a_ref[1]], axis=1)
        h = (1.0 + e_ref[0, 0]) * xb + ab
        z = jnp.maximum(jnp.dot(h, w1_ref[...],
                                preferred_element_type=jnp.float32)
                        + b1_ref[...], 0.0)
        o = jnp.dot(z, w2_ref[...], preferred_element_type=jnp.float32) \
            + b2_ref[...]
        o = jnp.maximum(o * s_ref[...] + be_ref[...], 0.0)
        o_ref[0] = o[:, : _H // 2]
        o_ref[1] = o[:, _H // 2:]

    xspec = (pl.BlockSpec((_BR, din), lambda i: (i, 0)) if first else
             pl.BlockSpec((2, _BR, _H // 2), lambda i: (0, i, 0)))
    return pl.pallas_call(
        body,
        grid=(_NB,),
        in_specs=[
            xspec,
            pl.BlockSpec((2, _BR, dh_pad), lambda i: (0, i, 0)),
            pl.BlockSpec((din, _H), lambda i: (0, 0)),
            pl.BlockSpec((1, _H), lambda i: (0, 0)),
            pl.BlockSpec((_H, _H), lambda i: (0, 0)),
            pl.BlockSpec((1, _H), lambda i: (0, 0)),
            pl.BlockSpec((1, _H), lambda i: (0, 0)),
            pl.BlockSpec((1, _H), lambda i: (0, 0)),
            pl.BlockSpec((1, 1), lambda i: (0, 0)),
        ],
        out_specs=pl.BlockSpec((2, _BR, _H // 2), lambda i: (0, i, 0)),
        out_shape=jax.ShapeDtypeStruct((2, _N, _H // 2), jnp.float32),
    )(xres, agg, w1, b1.reshape(1, _H), w2, b2.reshape(1, _H),
      scale.reshape(1, _H), beta.reshape(1, _H), eps.reshape(1, 1))


def _mlp_pool(xres, agg, w1, b1, w2, b2, scale, beta, eps, batch3d):
    def body(x_ref, a_ref, w1_ref, b1_ref, w2_ref, b2_ref, s_ref, be_ref,
             e_ref, bt_ref, p_ref, c_ref):
        i = pl.program_id(0)
        xb = jnp.concatenate([x_ref[0], x_ref[1]], axis=1)
        ab = jnp.concatenate([a_ref[0], a_ref[1]], axis=1)
        h = (1.0 + e_ref[0, 0]) * xb + ab
        z = jnp.maximum(jnp.dot(h, w1_ref[...],
                                preferred_element_type=jnp.float32)
                        + b1_ref[...], 0.0)
        o = jnp.dot(z, w2_ref[...], preferred_element_type=jnp.float32) \
            + b2_ref[...]
        o = jnp.maximum(o * s_ref[...] + be_ref[...], 0.0)
        seg = bt_ref[0, 0]
        onehot = (lax.broadcasted_iota(jnp.int32, (_BR, _NG), 1)
                  == seg[:, None]).astype(jnp.float32)
        psum = lax.dot_general(onehot, o, (((0,), (0,)), ((), ())),
                               preferred_element_type=jnp.float32)
        pcnt = lax.dot_general(onehot, jnp.ones((_BR, 8), jnp.float32),
                               (((0,), (0,)), ((), ())),
                               preferred_element_type=jnp.float32)

        @pl.when(i == 0)
        def _():
            p_ref[...] = jnp.zeros_like(p_ref)
            c_ref[...] = jnp.zeros_like(c_ref)

        p_ref[...] += psum
        c_ref[...] += pcnt

    return pl.pallas_call(
        body,
        grid=(_NB,),
        in_specs=[
            pl.BlockSpec((2, _BR, _H // 2), lambda i: (0, i, 0)),
            pl.BlockSpec((2, _BR, _H // 2), lambda i: (0, i, 0)),
            pl.BlockSpec((_H, _H), lambda i: (0, 0)),
            pl.BlockSpec((1, _H), lambda i: (0, 0)),
            pl.BlockSpec((_H, _H), lambda i: (0, 0)),
            pl.BlockSpec((1, _H), lambda i: (0, 0)),
            pl.BlockSpec((1, _H), lambda i: (0, 0)),
            pl.BlockSpec((1, _H), lambda i: (0, 0)),
            pl.BlockSpec((1, 1), lambda i: (0, 0)),
            pl.BlockSpec((1, 1, _BR), lambda i: (i, 0, 0)),
        ],
        out_specs=[
            pl.BlockSpec((_NG, _H), lambda i: (0, 0)),
            pl.BlockSpec((_NG, 8), lambda i: (0, 0)),
        ],
        out_shape=[
            jax.ShapeDtypeStruct((_NG, _H), jnp.float32),
            jax.ShapeDtypeStruct((_NG, 8), jnp.float32),
        ],
    )(xres, agg, w1, b1.reshape(1, _H), w2, b2.reshape(1, _H),
      scale.reshape(1, _H), beta.reshape(1, _H), eps.reshape(1, 1), batch3d)


def _classifier(pooled, cnt, w1, b1, w2, b2):
    def body(p_ref, c_ref, w1_ref, b1_ref, w2_ref, b2_ref, o_ref):
        mean = p_ref[...] / jnp.clip(c_ref[:, 0:1], 1.0)
        z = jnp.maximum(jnp.dot(mean, w1_ref[...],
                                preferred_element_type=jnp.float32)
                        + b1_ref[...], 0.0)
        o_ref[...] = jnp.dot(z, w2_ref[...],
                             preferred_element_type=jnp.float32) + b2_ref[...]

    return pl.pallas_call(
        body,
        out_shape=jax.ShapeDtypeStruct((_NG, 3), jnp.float32),
    )(pooled, cnt, w1, b1.reshape(1, _H), w2, b2.reshape(1, 3))


_msg16 = _make_msg_kernel(16)
_msg32 = _make_msg_kernel(32)


def kernel(x, edge_index, batch, edge_attr, params):
    src2d = edge_index[0].reshape(_E // _CH, _CH)
    dst2d = edge_index[1].reshape(_E // _CH, _CH)
    a2d = edge_attr.reshape(_E // _CH, _CH)
    batch3d = batch.reshape(_NB, 1, _BR)

    lys = params["layers"]
    bn_eps = 1e-5

    xt = jnp.zeros((_NCORES, _N, 16), jnp.float32)
    xt = xt.at[0, :, :2].set(x[:, :2]).at[1, :, :2].set(x[:, 2:4])
    w0 = lys[0]["edge_lin"]["W"][0]
    be0 = lys[0]["edge_lin"]["b"]
    wb0 = jnp.zeros((_NCORES, 2, 16), jnp.float32)
    wb0 = (wb0.at[0, 0, :2].set(w0[:2]).at[0, 1, :2].set(be0[:2])
              .at[1, 0, :2].set(w0[2:]).at[1, 1, :2].set(be0[2:]))
    agg0 = _msg16(xt, src2d, dst2d, a2d, wb0)
    s0 = lys[0]["bn_gamma"] / jnp.sqrt(1.0 + bn_eps)
    tbl1 = _mlp_mid(x, agg0, lys[0]["nn1"]["W"], lys[0]["nn1"]["b"],
                    lys[0]["nn2"]["W"], lys[0]["nn2"]["b"], s0,
                    lys[0]["bn_beta"], lys[0]["eps"], 4, 16)

    w1v = lys[1]["edge_lin"]["W"][0]
    b1v = lys[1]["edge_lin"]["b"]
    wb1 = jnp.stack([jnp.stack([w1v[:32], b1v[:32]]),
                     jnp.stack([w1v[32:], b1v[32:]])])
    agg1 = _msg32(tbl1, src2d, dst2d, a2d, wb1)
    s1 = lys[1]["bn_gamma"] / jnp.sqrt(1.0 + bn_eps)
    tbl2 = _mlp_mid(tbl1, agg1, lys[1]["nn1"]["W"], lys[1]["nn1"]["b"],
                    lys[1]["nn2"]["W"], lys[1]["nn2"]["b"], s1,
                    lys[1]["bn_beta"], lys[1]["eps"], _H, 32)

    w2v = lys[2]["edge_lin"]["W"][0]
    b2v = lys[2]["edge_lin"]["b"]
    wb2 = jnp.stack([jnp.stack([w2v[:32], b2v[:32]]),
                     jnp.stack([w2v[32:], b2v[32:]])])
    agg2 = _msg32(tbl2, src2d, dst2d, a2d, wb2)
    s2 = lys[2]["bn_gamma"] / jnp.sqrt(1.0 + bn_eps)
    pooled, cnt = _mlp_pool(tbl2, agg2, lys[2]["nn1"]["W"], lys[2]["nn1"]["b"],
                            lys[2]["nn2"]["W"], lys[2]["nn2"]["b"], s2,
                            lys[2]["bn_beta"], lys[2]["eps"], batch3d)

    cls = params["cls"]
    return _classifier(pooled, cnt, cls["l1"]["W"], cls["l1"]["b"],
                       cls["l2"]["W"], cls["l2"]["b"])


# speedup vs baseline: 3.2403x; 3.2403x over previous
import functools

import jax
import jax.numpy as jnp
from jax import lax
from jax.experimental import pallas as pl
from jax.experimental.pallas import tpu as pltpu
from jax.experimental.pallas import tpu_sc as plsc

_N = 50000
_E = 800000
_NG = 512
_H = 64

_NCORES = 2
_NTILES = 16
_CH = 125
_ROWS = _E // _CH
_SUB = 16
_TROWS = _ROWS // _NTILES
_SUPS = _TROWS // _SUB
_NP = 50048
_TSTR = _NP // _NTILES
_LSTR = _N - 15 * _TSTR
_ZR = 136


def _make_msg_kernel(dh):
    nreg = dh // 16
    mesh = plsc.VectorSubcoreMesh(core_axis_name="c", subcore_axis_name="s")

    @functools.partial(
        pl.kernel,
        out_type=jax.ShapeDtypeStruct((_NCORES, _N, dh), jnp.float32),
        mesh=mesh,
        scratch_types=[
            pltpu.VMEM((_SUB, _CH), jnp.int32),
            pltpu.VMEM((_SUB, _CH), jnp.int32),
            pltpu.VMEM((_SUB * _CH,), jnp.float32),
            pltpu.VMEM((_CH, dh), jnp.float32),
            pltpu.VMEM((2, dh), jnp.float32),
            pltpu.VMEM((_ZR, dh), jnp.float32),
            pltpu.VMEM_SHARED((_NP, dh), jnp.float32),
            pltpu.SemaphoreType.DMA,
        ],
        compiler_params=pltpu.CompilerParams(needs_layout_passes=False,
                                             use_tc_tiling_on_sc=False),
    )
    def msg(tbl0, tbl1, srcm, dstm, am, wb, out, srcb, dstb, ab, rows, wbv,
            zb, agg, gsem):
        c = lax.axis_index("c")
        t = lax.axis_index("s")
        zi = jnp.zeros((16,), jnp.int32)
        zf = jnp.zeros((16,), jnp.float32)

        pltpu.sync_copy(wb.at[c], wbv)

        def zrow(i, carry):
            for r in range(nreg):
                zb[i, pl.ds(r * 16, 16)] = zf
            return carry
        lax.fori_loop(0, _ZR, zrow, 0)

        def zcopy(j, carry):
            pltpu.sync_copy(zb, agg.at[pl.ds(t * _TSTR + j * _ZR, _ZR)])
            return carry
        lax.fori_loop(0, _TSTR // _ZR, zcopy, 0)
        plsc.subcore_barrier()

        wregs = [wbv[0, pl.ds(r * 16, 16)] for r in range(nreg)]
        bregs = [wbv[1, pl.ds(r * 16, 16)] for r in range(nreg)]

        def make_super_body(tbl):
            def super_body(s, carry):
                r0 = t * _TROWS + s * _SUB
                pltpu.sync_copy(srcm.at[pl.ds(r0, _SUB)], srcb)
                pltpu.sync_copy(dstm.at[pl.ds(r0, _SUB)], dstb)
                pltpu.sync_copy(am.at[pl.ds(r0 * _CH, _SUB * _CH)], ab)
                for kk in range(_SUB):
                    pltpu.async_copy(tbl.at[srcb.at[kk]], rows, gsem).wait()

                    def edge_body(e, cy):
                        av = plsc.load_gather(ab, [zi + (kk * _CH + e)])
                        for r in range(nreg):
                            xv = rows[e, pl.ds(r * 16, 16)]
                            rows[e, pl.ds(r * 16, 16)] = jnp.maximum(
                                xv + av * wregs[r] + bregs[r], 0.0)
                        return cy
                    lax.fori_loop(0, _CH, edge_body, 0)
                    pltpu.sync_copy(rows, agg.at[dstb.at[kk]], add=True)
                return carry
            return super_body

        @pl.when(c == 0)
        def _():
            lax.fori_loop(0, _SUPS, make_super_body(tbl0), 0)

        @pl.when(c == 1)
        def _():
            lax.fori_loop(0, _SUPS, make_super_body(tbl1), 0)

        plsc.subcore_barrier()

        @pl.when(t < _NTILES - 1)
        def _():
            pltpu.sync_copy(agg.at[pl.ds(t * _TSTR, _TSTR)],
                            out.at[c].at[pl.ds(t * _TSTR, _TSTR)])

        @pl.when(t == _NTILES - 1)
        def _():
            pltpu.sync_copy(agg.at[pl.ds(15 * _TSTR, _LSTR)],
                            out.at[c].at[pl.ds(15 * _TSTR, _LSTR)])

    return msg


_BR = 400
_NB = _N // _BR


def _mlp_mid(xres, agg, w1, b1, w2, b2, scale, beta, eps, din, dh_pad):
    first = din != _H

    def body(x0_ref, x1_ref, a_ref, w1_ref, b1_ref, w2_ref, b2_ref, s_ref,
             be_ref, e_ref, o0_ref, o1_ref):
        if first:
            xb = x0_ref[...]
            ab = jnp.concatenate(
                [a_ref[0, :, : din // 2], a_ref[1, :, : din // 2]], axis=1)
        else:
            xb = jnp.concatenate([x0_ref[...], x1_ref[...]], axis=1)
            ab = jnp.concatenate([a_ref[0], a_ref[1]], axis=1)
        h = (1.0 + e_ref[0, 0]) * xb + ab
        z = jnp.maximum(jnp.dot(h, w1_ref[...],
                                preferred_element_type=jnp.float32)
                        + b1_ref[...], 0.0)
        o = jnp.dot(z, w2_ref[...], preferred_element_type=jnp.float32) \
            + b2_ref[...]
        o = jnp.maximum(o * s_ref[...] + be_ref[...], 0.0)
        o0_ref[...] = o[:, : _H // 2]
        o1_ref[...] = o[:, _H // 2:]

    if first:
        x0, x1 = xres, xres
        xspecs = [pl.BlockSpec((_BR, din), lambda i: (i, 0)),
                  pl.BlockSpec((_BR, din), lambda i: (i, 0))]
    else:
        x0, x1 = xres
        xspecs = [pl.BlockSpec((_BR, _H // 2), lambda i: (i, 0)),
                  pl.BlockSpec((_BR, _H // 2), lambda i: (i, 0))]
    return pl.pallas_call(
        body,
        grid=(_NB,),
        in_specs=xspecs + [
            pl.BlockSpec((2, _BR, dh_pad), lambda i: (0, i, 0)),
            pl.BlockSpec((din, _H), lambda i: (0, 0)),
            pl.BlockSpec((1, _H), lambda i: (0, 0)),
            pl.BlockSpec((_H, _H), lambda i: (0, 0)),
            pl.BlockSpec((1, _H), lambda i: (0, 0)),
            pl.BlockSpec((1, _H), lambda i: (0, 0)),
            pl.BlockSpec((1, _H), lambda i: (0, 0)),
            pl.BlockSpec((1, 1), lambda i: (0, 0)),
        ],
        out_specs=[pl.BlockSpec((_BR, _H // 2), lambda i: (i, 0)),
                   pl.BlockSpec((_BR, _H // 2), lambda i: (i, 0))],
        out_shape=[jax.ShapeDtypeStruct((_N, _H // 2), jnp.float32),
                   jax.ShapeDtypeStruct((_N, _H // 2), jnp.float32)],
    )(x0, x1, agg, w1, b1.reshape(1, _H), w2, b2.reshape(1, _H),
      scale.reshape(1, _H), beta.reshape(1, _H), eps.reshape(1, 1))


def _mlp_pool(xres, agg, w1, b1, w2, b2, scale, beta, eps, batch3d):
    def body(x0_ref, x1_ref, a_ref, w1_ref, b1_ref, w2_ref, b2_ref, s_ref,
             be_ref, e_ref, bt_ref, p_ref, c_ref):
        i = pl.program_id(0)
        xb = jnp.concatenate([x0_ref[...], x1_ref[...]], axis=1)
        ab = jnp.concatenate([a_ref[0], a_ref[1]], axis=1)
        h = (1.0 + e_ref[0, 0]) * xb + ab
        z = jnp.maximum(jnp.dot(h, w1_ref[...],
                                preferred_element_type=jnp.float32)
                        + b1_ref[...], 0.0)
        o = jnp.dot(z, w2_ref[...], preferred_element_type=jnp.float32) \
            + b2_ref[...]
        o = jnp.maximum(o * s_ref[...] + be_ref[...], 0.0)
        seg = bt_ref[0, 0]
        onehot = (lax.broadcasted_iota(jnp.int32, (_BR, _NG), 1)
                  == seg[:, None]).astype(jnp.float32)
        psum = lax.dot_general(onehot, o, (((0,), (0,)), ((), ())),
                               preferred_element_type=jnp.float32)
        pcnt = lax.dot_general(onehot, jnp.ones((_BR, 8), jnp.float32),
                               (((0,), (0,)), ((), ())),
                               preferred_element_type=jnp.float32)

        @pl.when(i == 0)
        def _():
            p_ref[...] = jnp.zeros_like(p_ref)
            c_ref[...] = jnp.zeros_like(c_ref)

        p_ref[...] += psum
        c_ref[...] += pcnt

    return pl.pallas_call(
        body,
        grid=(_NB,),
        in_specs=[
            pl.BlockSpec((_BR, _H // 2), lambda i: (i, 0)),
            pl.BlockSpec((_BR, _H // 2), lambda i: (i, 0)),
            pl.BlockSpec((2, _BR, _H // 2), lambda i: (0, i, 0)),
            pl.BlockSpec((_H, _H), lambda i: (0, 0)),
            pl.BlockSpec((1, _H), lambda i: (0, 0)),
            pl.BlockSpec((_H, _H), lambda i: (0, 0)),
            pl.BlockSpec((1, _H), lambda i: (0, 0)),
            pl.BlockSpec((1, _H), lambda i: (0, 0)),
            pl.BlockSpec((1, _H), lambda i: (0, 0)),
            pl.BlockSpec((1, 1), lambda i: (0, 0)),
            pl.BlockSpec((1, 1, _BR), lambda i: (i, 0, 0)),
        ],
        out_specs=[
            pl.BlockSpec((_NG, _H), lambda i: (0, 0)),
            pl.BlockSpec((_NG, 8), lambda i: (0, 0)),
        ],
        out_shape=[
            jax.ShapeDtypeStruct((_NG, _H), jnp.float32),
            jax.ShapeDtypeStruct((_NG, 8), jnp.float32),
        ],
    )(xres[0], xres[1], agg, w1, b1.reshape(1, _H), w2, b2.reshape(1, _H),
      scale.reshape(1, _H), beta.reshape(1, _H), eps.reshape(1, 1), batch3d)


def _classifier(pooled, cnt, w1, b1, w2, b2):
    def body(p_ref, c_ref, w1_ref, b1_ref, w2_ref, b2_ref, o_ref):
        mean = p_ref[...] / jnp.clip(c_ref[:, 0:1], 1.0)
        z = jnp.maximum(jnp.dot(mean, w1_ref[...],
                                preferred_element_type=jnp.float32)
                        + b1_ref[...], 0.0)
        o_ref[...] = jnp.dot(z, w2_ref[...],
                             preferred_element_type=jnp.float32) + b2_ref[...]

    return pl.pallas_call(
        body,
        out_shape=jax.ShapeDtypeStruct((_NG, 3), jnp.float32),
    )(pooled, cnt, w1, b1.reshape(1, _H), w2, b2.reshape(1, 3))


_msg16 = _make_msg_kernel(16)
_msg32 = _make_msg_kernel(32)


def kernel(x, edge_index, batch, edge_attr, params):
    src2d = edge_index[0].reshape(_E // _CH, _CH)
    dst2d = edge_index[1].reshape(_E // _CH, _CH)
    a2d = edge_attr.reshape(_E)
    batch3d = batch.reshape(_NB, 1, _BR)

    lys = params["layers"]
    bn_eps = 1e-5

    xt0 = jnp.pad(x[:, :2], ((0, 0), (0, 14)))
    xt1 = jnp.pad(x[:, 2:4], ((0, 0), (0, 14)))
    w0 = lys[0]["edge_lin"]["W"][0]
    be0 = lys[0]["edge_lin"]["b"]
    wb0 = jnp.zeros((_NCORES, 2, 16), jnp.float32)
    wb0 = (wb0.at[0, 0, :2].set(w0[:2]).at[0, 1, :2].set(be0[:2])
              .at[1, 0, :2].set(w0[2:]).at[1, 1, :2].set(be0[2:]))
    agg0 = _msg16(xt0, xt1, src2d, dst2d, a2d, wb0)
    s0 = lys[0]["bn_gamma"] / jnp.sqrt(1.0 + bn_eps)
    tbl1 = _mlp_mid(x, agg0, lys[0]["nn1"]["W"], lys[0]["nn1"]["b"],
                    lys[0]["nn2"]["W"], lys[0]["nn2"]["b"], s0,
                    lys[0]["bn_beta"], lys[0]["eps"], 4, 16)

    w1v = lys[1]["edge_lin"]["W"][0]
    b1v = lys[1]["edge_lin"]["b"]
    wb1 = jnp.stack([jnp.stack([w1v[:32], b1v[:32]]),
                     jnp.stack([w1v[32:], b1v[32:]])])
    agg1 = _msg32(tbl1[0], tbl1[1], src2d, dst2d, a2d, wb1)
    s1 = lys[1]["bn_gamma"] / jnp.sqrt(1.0 + bn_eps)
    tbl2 = _mlp_mid(tbl1, agg1, lys[1]["nn1"]["W"], lys[1]["nn1"]["b"],
                    lys[1]["nn2"]["W"], lys[1]["nn2"]["b"], s1,
                    lys[1]["bn_beta"], lys[1]["eps"], _H, 32)

    w2v = lys[2]["edge_lin"]["W"][0]
    b2v = lys[2]["edge_lin"]["b"]
    wb2 = jnp.stack([jnp.stack([w2v[:32], b2v[:32]]),
                     jnp.stack([w2v[32:], b2v[32:]])])
    agg2 = _msg32(tbl2[0], tbl2[1], src2d, dst2d, a2d, wb2)
    s2 = lys[2]["bn_gamma"] / jnp.sqrt(1.0 + bn_eps)
    pooled, cnt = _mlp_pool(tbl2, agg2, lys[2]["nn1"]["W"], lys[2]["nn1"]["b"],
                            lys[2]["nn2"]["W"], lys[2]["nn2"]["b"], s2,
                            lys[2]["bn_beta"], lys[2]["eps"], batch3d)

    cls = params["cls"]
    return _classifier(pooled, cnt, cls["l1"]["W"], cls["l1"]["b"],
                       cls["l2"]["W"], cls["l2"]["b"])


# speedup vs baseline: 5.7710x; 1.7810x over previous
import functools

import jax
import jax.numpy as jnp
from jax import lax
from jax.experimental import pallas as pl
from jax.experimental.pallas import tpu as pltpu
from jax.experimental.pallas import tpu_sc as plsc

_N = 50000
_E = 800000
_NG = 512
_H = 64

_NCORES = 2
_NTILES = 16
_CH = 125
_ROWS = _E // _CH
_SUB = 16
_TROWS = _ROWS // _NTILES
_SUPS = _TROWS // _SUB
_NP = 50048
_TSTR = _NP // _NTILES
_LSTR = _N - 15 * _TSTR
_ZR = 136


def _make_msg_kernel(dh):
    nreg = dh // 16
    mesh = plsc.VectorSubcoreMesh(core_axis_name="c", subcore_axis_name="s")

    @functools.partial(
        pl.kernel,
        out_type=jax.ShapeDtypeStruct((_NCORES, _N, dh), jnp.float32),
        mesh=mesh,
        scratch_types=[
            pltpu.VMEM((_SUB, _CH), jnp.int32),
            pltpu.VMEM((_SUB, _CH), jnp.int32),
            pltpu.VMEM((_SUB * _CH,), jnp.float32),
            pltpu.VMEM((_CH, dh), jnp.float32),
            pltpu.VMEM((_CH, dh), jnp.float32),
            pltpu.VMEM((2, dh), jnp.float32),
            pltpu.VMEM((_ZR, dh), jnp.float32),
            pltpu.VMEM_SHARED((_NP, dh), jnp.float32),
            pltpu.SemaphoreType.DMA,
            pltpu.SemaphoreType.DMA,
            pltpu.SemaphoreType.DMA,
            pltpu.SemaphoreType.DMA,
        ],
        compiler_params=pltpu.CompilerParams(needs_layout_passes=False,
                                             use_tc_tiling_on_sc=False),
    )
    def msg(tbl0, tbl1, srcm, dstm, am, wb, out, srcb, dstb, ab, rows0, rows1,
            wbv, zb, agg, gsem0, gsem1, ssem0, ssem1):
        c = lax.axis_index("c")
        t = lax.axis_index("s")
        zi = jnp.zeros((16,), jnp.int32)
        zf = jnp.zeros((16,), jnp.float32)

        pltpu.sync_copy(wb.at[c], wbv)

        def zrow(i, carry):
            for r in range(nreg):
                zb[i, pl.ds(r * 16, 16)] = zf
            return carry
        lax.fori_loop(0, _ZR, zrow, 0)

        def zcopy(j, carry):
            pltpu.sync_copy(zb, agg.at[pl.ds(t * _TSTR + j * _ZR, _ZR)])
            return carry
        lax.fori_loop(0, _TSTR // _ZR, zcopy, 0)
        plsc.subcore_barrier()

        wregs = [wbv[0, pl.ds(r * 16, 16)] for r in range(nreg)]

        bufs = (rows0, rows1)
        gsems = (gsem0, gsem1)
        ssems = (ssem0, ssem1)

        def make_super_body(tbl):
            def super_body(s, carry):
                r0 = t * _TROWS + s * _SUB
                pltpu.sync_copy(srcm.at[pl.ds(r0, _SUB)], srcb)
                pltpu.sync_copy(dstm.at[pl.ds(r0, _SUB)], dstb)
                pltpu.sync_copy(am.at[pl.ds(r0 * _CH, _SUB * _CH)], ab)
                gd = [None, None]
                sd = [None, None]
                gd[0] = pltpu.async_copy(tbl.at[srcb.at[0]], bufs[0], gsems[0])
                for kk in range(_SUB):
                    b = kk % 2
                    ob = 1 - b
                    gd[b].wait()
                    if kk + 1 < _SUB:
                        if sd[ob] is not None:
                            sd[ob].wait()
                        gd[ob] = pltpu.async_copy(tbl.at[srcb.at[kk + 1]],
                                                  bufs[ob], gsems[ob])

                    def edge_body(e):
                        av = plsc.load_gather(ab, [zi + (kk * _CH + e)])
                        for r in range(nreg):
                            xv = bufs[b][e, pl.ds(r * 16, 16)]
                            bufs[b][e, pl.ds(r * 16, 16)] = jnp.maximum(
                                xv + av * wregs[r], 0.0)
                    plsc.parallel_loop(0, _CH, unroll=5)(edge_body)
                    sd[b] = pltpu.async_copy(bufs[b], agg.at[dstb.at[kk]],
                                             ssems[b], add=True)
                sd[0].wait()
                sd[1].wait()
                return carry
            return super_body

        @pl.when(c == 0)
        def _():
            lax.fori_loop(0, _SUPS, make_super_body(tbl0), 0)

        @pl.when(c == 1)
        def _():
            lax.fori_loop(0, _SUPS, make_super_body(tbl1), 0)

        plsc.subcore_barrier()

        @pl.when(t < _NTILES - 1)
        def _():
            pltpu.sync_copy(agg.at[pl.ds(t * _TSTR, _TSTR)],
                            out.at[c].at[pl.ds(t * _TSTR, _TSTR)])

        @pl.when(t == _NTILES - 1)
        def _():
            pltpu.sync_copy(agg.at[pl.ds(15 * _TSTR, _LSTR)],
                            out.at[c].at[pl.ds(15 * _TSTR, _LSTR)])

    return msg


_BR = 400
_NB = _N // _BR


def _mlp_mid(xres, agg, w1, b1, w2, b2, scale, beta, eps, bprev, bnext, din,
             dh_pad):
    first = din != _H

    def body(x0_ref, x1_ref, a_ref, w1_ref, b1_ref, w2_ref, b2_ref, s_ref,
             be_ref, e_ref, bn_ref, bp_ref, o0_ref, o1_ref):
        if first:
            xb = x0_ref[...]
            ab = jnp.concatenate(
                [a_ref[0, :, : din // 2], a_ref[1, :, : din // 2]], axis=1)
        else:
            xb = jnp.concatenate([x0_ref[...], x1_ref[...]], axis=1) \
                - bp_ref[...]
            ab = jnp.concatenate([a_ref[0], a_ref[1]], axis=1)
        h = (1.0 + e_ref[0, 0]) * xb + ab
        z = jnp.maximum(jnp.dot(h, w1_ref[...],
                                preferred_element_type=jnp.float32)
                        + b1_ref[...], 0.0)
        o = jnp.dot(z, w2_ref[...], preferred_element_type=jnp.float32) \
            + b2_ref[...]
        o = jnp.maximum(o * s_ref[...] + be_ref[...], 0.0) + bn_ref[...]
        o0_ref[...] = o[:, : _H // 2]
        o1_ref[...] = o[:, _H // 2:]

    if first:
        x0, x1 = xres, xres
        xspecs = [pl.BlockSpec((_BR, din), lambda i: (i, 0)),
                  pl.BlockSpec((_BR, din), lambda i: (i, 0))]
    else:
        x0, x1 = xres
        xspecs = [pl.BlockSpec((_BR, _H // 2), lambda i: (i, 0)),
                  pl.BlockSpec((_BR, _H // 2), lambda i: (i, 0))]
    return pl.pallas_call(
        body,
        grid=(_NB,),
        in_specs=xspecs + [
            pl.BlockSpec((2, _BR, dh_pad), lambda i: (0, i, 0)),
            pl.BlockSpec((din, _H), lambda i: (0, 0)),
            pl.BlockSpec((1, _H), lambda i: (0, 0)),
            pl.BlockSpec((_H, _H), lambda i: (0, 0)),
            pl.BlockSpec((1, _H), lambda i: (0, 0)),
            pl.BlockSpec((1, _H), lambda i: (0, 0)),
            pl.BlockSpec((1, _H), lambda i: (0, 0)),
            pl.BlockSpec((1, 1), lambda i: (0, 0)),
            pl.BlockSpec((1, _H), lambda i: (0, 0)),
            pl.BlockSpec((1, _H), lambda i: (0, 0)),
        ],
        out_specs=[pl.BlockSpec((_BR, _H // 2), lambda i: (i, 0)),
                   pl.BlockSpec((_BR, _H // 2), lambda i: (i, 0))],
        out_shape=[jax.ShapeDtypeStruct((_N, _H // 2), jnp.float32),
                   jax.ShapeDtypeStruct((_N, _H // 2), jnp.float32)],
    )(x0, x1, agg, w1, b1.reshape(1, _H), w2, b2.reshape(1, _H),
      scale.reshape(1, _H), beta.reshape(1, _H), eps.reshape(1, 1),
      bnext.reshape(1, _H), bprev.reshape(1, _H))


def _mlp_pool(xres, agg, w1, b1, w2, b2, scale, beta, eps, bprev, batch3d):
    def body(x0_ref, x1_ref, a_ref, w1_ref, b1_ref, w2_ref, b2_ref, s_ref,
             be_ref, e_ref, bp_ref, bt_ref, p_ref, c_ref):
        i = pl.program_id(0)
        xb = jnp.concatenate([x0_ref[...], x1_ref[...]], axis=1) - bp_ref[...]
        ab = jnp.concatenate([a_ref[0], a_ref[1]], axis=1)
        h = (1.0 + e_ref[0, 0]) * xb + ab
        z = jnp.maximum(jnp.dot(h, w1_ref[...],
                                preferred_element_type=jnp.float32)
                        + b1_ref[...], 0.0)
        o = jnp.dot(z, w2_ref[...], preferred_element_type=jnp.float32) \
            + b2_ref[...]
        o = jnp.maximum(o * s_ref[...] + be_ref[...], 0.0)
        seg = bt_ref[0, 0]
        onehot = (lax.broadcasted_iota(jnp.int32, (_BR, _NG), 1)
                  == seg[:, None]).astype(jnp.float32)
        psum = lax.dot_general(onehot, o, (((0,), (0,)), ((), ())),
                               preferred_element_type=jnp.float32)
        pcnt = lax.dot_general(onehot, jnp.ones((_BR, 8), jnp.float32),
                               (((0,), (0,)), ((), ())),
                               preferred_element_type=jnp.float32)

        @pl.when(i == 0)
        def _():
            p_ref[...] = jnp.zeros_like(p_ref)
            c_ref[...] = jnp.zeros_like(c_ref)

        p_ref[...] += psum
        c_ref[...] += pcnt

    return pl.pallas_call(
        body,
        grid=(_NB,),
        in_specs=[
            pl.BlockSpec((_BR, _H // 2), lambda i: (i, 0)),
            pl.BlockSpec((_BR, _H // 2), lambda i: (i, 0)),
            pl.BlockSpec((2, _BR, _H // 2), lambda i: (0, i, 0)),
            pl.BlockSpec((_H, _H), lambda i: (0, 0)),
            pl.BlockSpec((1, _H), lambda i: (0, 0)),
            pl.BlockSpec((_H, _H), lambda i: (0, 0)),
            pl.BlockSpec((1, _H), lambda i: (0, 0)),
            pl.BlockSpec((1, _H), lambda i: (0, 0)),
            pl.BlockSpec((1, _H), lambda i: (0, 0)),
            pl.BlockSpec((1, 1), lambda i: (0, 0)),
            pl.BlockSpec((1, _H), lambda i: (0, 0)),
            pl.BlockSpec((1, 1, _BR), lambda i: (i, 0, 0)),
        ],
        out_specs=[
            pl.BlockSpec((_NG, _H), lambda i: (0, 0)),
            pl.BlockSpec((_NG, 8), lambda i: (0, 0)),
        ],
        out_shape=[
            jax.ShapeDtypeStruct((_NG, _H), jnp.float32),
            jax.ShapeDtypeStruct((_NG, 8), jnp.float32),
        ],
    )(xres[0], xres[1], agg, w1, b1.reshape(1, _H), w2, b2.reshape(1, _H),
      scale.reshape(1, _H), beta.reshape(1, _H), eps.reshape(1, 1),
      bprev.reshape(1, _H), batch3d)


def _classifier(pooled, cnt, w1, b1, w2, b2):
    def body(p_ref, c_ref, w1_ref, b1_ref, w2_ref, b2_ref, o_ref):
        mean = p_ref[...] / jnp.clip(c_ref[:, 0:1], 1.0)
        z = jnp.maximum(jnp.dot(mean, w1_ref[...],
                                preferred_element_type=jnp.float32)
                        + b1_ref[...], 0.0)
        o_ref[...] = jnp.dot(z, w2_ref[...],
                             preferred_element_type=jnp.float32) + b2_ref[...]

    return pl.pallas_call(
        body,
        out_shape=jax.ShapeDtypeStruct((_NG, 3), jnp.float32),
    )(pooled, cnt, w1, b1.reshape(1, _H), w2, b2.reshape(1, 3))


_msg16 = _make_msg_kernel(16)
_msg32 = _make_msg_kernel(32)


def kernel(x, edge_index, batch, edge_attr, params):
    src2d = edge_index[0].reshape(_E // _CH, _CH)
    dst2d = edge_index[1].reshape(_E // _CH, _CH)
    a2d = edge_attr.reshape(_E)
    batch3d = batch.reshape(_NB, 1, _BR)

    lys = params["layers"]
    bn_eps = 1e-5

    be0 = lys[0]["edge_lin"]["b"]
    xt0 = jnp.pad(x[:, :2] + be0[:2], ((0, 0), (0, 14)))
    xt1 = jnp.pad(x[:, 2:4] + be0[2:], ((0, 0), (0, 14)))
    w0 = lys[0]["edge_lin"]["W"][0]
    wb0 = jnp.zeros((_NCORES, 2, 16), jnp.float32)
    wb0 = wb0.at[0, 0, :2].set(w0[:2]).at[1, 0, :2].set(w0[2:])
    agg0 = _msg16(xt0, xt1, src2d, dst2d, a2d, wb0)
    b1v = lys[1]["edge_lin"]["b"]
    b2v = lys[2]["edge_lin"]["b"]
    s0 = lys[0]["bn_gamma"] / jnp.sqrt(1.0 + bn_eps)
    tbl1 = _mlp_mid(x, agg0, lys[0]["nn1"]["W"], lys[0]["nn1"]["b"],
                    lys[0]["nn2"]["W"], lys[0]["nn2"]["b"], s0,
                    lys[0]["bn_beta"], lys[0]["eps"],
                    jnp.zeros((_H,), jnp.float32), b1v, 4, 16)

    w1v = lys[1]["edge_lin"]["W"][0]
    wb1 = jnp.stack([jnp.stack([w1v[:32], w1v[:32]]),
                     jnp.stack([w1v[32:], w1v[32:]])])
    agg1 = _msg32(tbl1[0], tbl1[1], src2d, dst2d, a2d, wb1)
    s1 = lys[1]["bn_gamma"] / jnp.sqrt(1.0 + bn_eps)
    tbl2 = _mlp_mid(tbl1, agg1, lys[1]["nn1"]["W"], lys[1]["nn1"]["b"],
                    lys[1]["nn2"]["W"], lys[1]["nn2"]["b"], s1,
                    lys[1]["bn_beta"], lys[1]["eps"], b1v, b2v, _H, 32)

    w2v = lys[2]["edge_lin"]["W"][0]
    wb2 = jnp.stack([jnp.stack([w2v[:32], w2v[:32]]),
                     jnp.stack([w2v[32:], w2v[32:]])])
    agg2 = _msg32(tbl2[0], tbl2[1], src2d, dst2d, a2d, wb2)
    s2 = lys[2]["bn_gamma"] / jnp.sqrt(1.0 + bn_eps)
    pooled, cnt = _mlp_pool(tbl2, agg2, lys[2]["nn1"]["W"], lys[2]["nn1"]["b"],
                            lys[2]["nn2"]["W"], lys[2]["nn2"]["b"], s2,
                            lys[2]["bn_beta"], lys[2]["eps"], b2v, batch3d)

    cls = params["cls"]
    return _classifier(pooled, cnt, cls["l1"]["W"], cls["l1"]["b"],
                       cls["l2"]["W"], cls["l2"]["b"])


# speedup vs baseline: 6.7147x; 1.1635x over previous
import functools

import jax
import jax.numpy as jnp
from jax import lax
from jax.experimental import pallas as pl
from jax.experimental.pallas import tpu as pltpu
from jax.experimental.pallas import tpu_sc as plsc

_N = 50000
_E = 800000
_NG = 512
_H = 64

_NCORES = 2
_NTILES = 16
_CH = 125
_ROWS = _E // _CH
_SUB = 16
_TROWS = _ROWS // _NTILES
_SUPS = _TROWS // _SUB
_NP = 50048
_TSTR = _NP // _NTILES
_LSTR = _N - 15 * _TSTR
_ZR = 136


def _make_msg_kernel(dh):
    nreg = dh // 16
    mesh = plsc.VectorSubcoreMesh(core_axis_name="c", subcore_axis_name="s")

    @functools.partial(
        pl.kernel,
        out_type=jax.ShapeDtypeStruct((_NCORES, _N, dh), jnp.float32),
        mesh=mesh,
        scratch_types=[
            pltpu.VMEM((_SUB, _CH), jnp.int32),
            pltpu.VMEM((_SUB, _CH), jnp.int32),
            pltpu.VMEM((_SUB * _CH,), jnp.float32),
            pltpu.VMEM((_CH, dh), jnp.float32),
            pltpu.VMEM((_CH, dh), jnp.float32),
            pltpu.VMEM((2, dh), jnp.float32),
            pltpu.VMEM((_ZR, dh), jnp.float32),
            pltpu.VMEM_SHARED((_NP, dh), jnp.float32),
            pltpu.SemaphoreType.DMA,
            pltpu.SemaphoreType.DMA,
            pltpu.SemaphoreType.DMA,
            pltpu.SemaphoreType.DMA,
        ],
        compiler_params=pltpu.CompilerParams(needs_layout_passes=False,
                                             use_tc_tiling_on_sc=False),
    )
    def msg(tbl0, tbl1, srcm, dstm, am, wb, out, srcb, dstb, ab, rows0, rows1,
            wbv, zb, agg, gsem0, gsem1, ssem0, ssem1):
        c = lax.axis_index("c")
        t = lax.axis_index("s")
        zi = jnp.zeros((16,), jnp.int32)
        zf = jnp.zeros((16,), jnp.float32)

        pltpu.sync_copy(wb.at[c], wbv)

        def zrow(i, carry):
            for r in range(nreg):
                zb[i, pl.ds(r * 16, 16)] = zf
            return carry
        lax.fori_loop(0, _ZR, zrow, 0)

        def zcopy(j, carry):
            pltpu.sync_copy(zb, agg.at[pl.ds(t * _TSTR + j * _ZR, _ZR)])
            return carry
        lax.fori_loop(0, _TSTR // _ZR, zcopy, 0)
        plsc.subcore_barrier()

        wregs = [wbv[0, pl.ds(r * 16, 16)] for r in range(nreg)]

        bufs = (rows0, rows1)
        gsems = (gsem0, gsem1)
        ssems = (ssem0, ssem1)

        def make_super_body(tbl):
            def super_body(s, carry):
                r0 = t * _TROWS + s * _SUB
                pltpu.sync_copy(srcm.at[pl.ds(r0, _SUB)], srcb)
                pltpu.sync_copy(dstm.at[pl.ds(r0, _SUB)], dstb)
                pltpu.sync_copy(am.at[pl.ds(r0 * _CH, _SUB * _CH)], ab)
                gd = [None, None]
                sd = [None, None]
                gd[0] = pltpu.async_copy(tbl.at[srcb.at[0]], bufs[0], gsems[0])
                for kk in range(_SUB):
                    b = kk % 2
                    ob = 1 - b
                    gd[b].wait()
                    if kk + 1 < _SUB:
                        if sd[ob] is not None:
                            sd[ob].wait()
                        gd[ob] = pltpu.async_copy(tbl.at[srcb.at[kk + 1]],
                                                  bufs[ob], gsems[ob])

                    def edge_body(e):
                        av = plsc.load_gather(ab, [zi + (kk * _CH + e)])
                        for r in range(nreg):
                            xv = bufs[b][e, pl.ds(r * 16, 16)]
                            bufs[b][e, pl.ds(r * 16, 16)] = jnp.maximum(
                                xv + av * wregs[r], 0.0)
                    plsc.parallel_loop(0, _CH, unroll=5)(edge_body)
                    sd[b] = pltpu.async_copy(bufs[b], agg.at[dstb.at[kk]],
                                             ssems[b], add=True)
                sd[0].wait()
                sd[1].wait()
                return carry
            return super_body

        @pl.when(c == 0)
        def _():
            lax.fori_loop(0, _SUPS, make_super_body(tbl0), 0)

        @pl.when(c == 1)
        def _():
            lax.fori_loop(0, _SUPS, make_super_body(tbl1), 0)

        plsc.subcore_barrier()

        @pl.when(t < _NTILES - 1)
        def _():
            pltpu.sync_copy(agg.at[pl.ds(t * _TSTR, _TSTR)],
                            out.at[c].at[pl.ds(t * _TSTR, _TSTR)])

        @pl.when(t == _NTILES - 1)
        def _():
            pltpu.sync_copy(agg.at[pl.ds(15 * _TSTR, _LSTR)],
                            out.at[c].at[pl.ds(15 * _TSTR, _LSTR)])

    return msg


_BR = 400
_NB = _N // _BR


def _mlp_mid(xres, agg, w1, b1, w2, b2, scale, beta, eps, bprev, bnext, din,
             dh_pad):
    first = din != _H

    def body(x0_ref, x1_ref, a_ref, w1_ref, b1_ref, w2_ref, b2_ref, s_ref,
             be_ref, e_ref, bn_ref, bp_ref, o0_ref, o1_ref):
        if first:
            xb = x0_ref[...]
            ab = jnp.concatenate(
                [a_ref[0, :, : din // 2], a_ref[1, :, : din // 2]], axis=1)
        else:
            xb = jnp.concatenate([x0_ref[...], x1_ref[...]], axis=1) \
                - bp_ref[...]
            ab = jnp.concatenate([a_ref[0], a_ref[1]], axis=1)
        h = (1.0 + e_ref[0, 0]) * xb + ab
        z = jnp.maximum(jnp.dot(h, w1_ref[...],
                                preferred_element_type=jnp.float32)
                        + b1_ref[...], 0.0)
        o = jnp.dot(z, w2_ref[...], preferred_element_type=jnp.float32) \
            + b2_ref[...]
        o = jnp.maximum(o * s_ref[...] + be_ref[...], 0.0) + bn_ref[...]
        o0_ref[...] = o[:, : _H // 2]
        o1_ref[...] = o[:, _H // 2:]

    if first:
        x0, x1 = xres, xres
        xspecs = [pl.BlockSpec((_BR, din), lambda i: (i, 0)),
                  pl.BlockSpec((_BR, din), lambda i: (i, 0))]
    else:
        x0, x1 = xres
        xspecs = [pl.BlockSpec((_BR, _H // 2), lambda i: (i, 0)),
                  pl.BlockSpec((_BR, _H // 2), lambda i: (i, 0))]
    return pl.pallas_call(
        body,
        grid=(_NB,),
        in_specs=xspecs + [
            pl.BlockSpec((2, _BR, dh_pad), lambda i: (0, i, 0)),
            pl.BlockSpec((din, _H), lambda i: (0, 0)),
            pl.BlockSpec((1, _H), lambda i: (0, 0)),
            pl.BlockSpec((_H, _H), lambda i: (0, 0)),
            pl.BlockSpec((1, _H), lambda i: (0, 0)),
            pl.BlockSpec((1, _H), lambda i: (0, 0)),
            pl.BlockSpec((1, _H), lambda i: (0, 0)),
            pl.BlockSpec((1, 1), lambda i: (0, 0)),
            pl.BlockSpec((1, _H), lambda i: (0, 0)),
            pl.BlockSpec((1, _H), lambda i: (0, 0)),
        ],
        out_specs=[pl.BlockSpec((_BR, _H // 2), lambda i: (i, 0)),
                   pl.BlockSpec((_BR, _H // 2), lambda i: (i, 0))],
        out_shape=[jax.ShapeDtypeStruct((_N, _H // 2), jnp.float32),
                   jax.ShapeDtypeStruct((_N, _H // 2), jnp.float32)],
    )(x0, x1, agg, w1, b1.reshape(1, _H), w2, b2.reshape(1, _H),
      scale.reshape(1, _H), beta.reshape(1, _H), eps.reshape(1, 1),
      bnext.reshape(1, _H), bprev.reshape(1, _H))


def _mlp_pool(xres, agg, w1, b1, w2, b2, scale, beta, eps, bprev, batch3d,
              cw1, cb1, cw2, cb2):
    def body(x0_ref, x1_ref, a_ref, w1_ref, b1_ref, w2_ref, b2_ref, s_ref,
             be_ref, e_ref, bp_ref, bt_ref, cw1_ref, cb1_ref, cw2_ref,
             cb2_ref, o_ref, p_ref, c_ref):
        i = pl.program_id(0)
        xb = jnp.concatenate([x0_ref[...], x1_ref[...]], axis=1) - bp_ref[...]
        ab = jnp.concatenate([a_ref[0], a_ref[1]], axis=1)
        h = (1.0 + e_ref[0, 0]) * xb + ab
        z = jnp.maximum(jnp.dot(h, w1_ref[...],
                                preferred_element_type=jnp.float32)
                        + b1_ref[...], 0.0)
        o = jnp.dot(z, w2_ref[...], preferred_element_type=jnp.float32) \
            + b2_ref[...]
        o = jnp.maximum(o * s_ref[...] + be_ref[...], 0.0)
        seg = bt_ref[0, 0]
        onehot = (lax.broadcasted_iota(jnp.int32, (_BR, _NG), 1)
                  == seg[:, None]).astype(jnp.float32)
        psum = lax.dot_general(onehot, o, (((0,), (0,)), ((), ())),
                               preferred_element_type=jnp.float32)
        pcnt = lax.dot_general(onehot, jnp.ones((_BR, 8), jnp.float32),
                               (((0,), (0,)), ((), ())),
                               preferred_element_type=jnp.float32)

        @pl.when(i == 0)
        def _():
            p_ref[...] = jnp.zeros_like(p_ref)
            c_ref[...] = jnp.zeros_like(c_ref)

        p_ref[...] += psum
        c_ref[...] += pcnt

        @pl.when(i == _NB - 1)
        def _():
            mean = p_ref[...] / jnp.clip(c_ref[:, 0:1], 1.0)
            zc = jnp.maximum(jnp.dot(mean, cw1_ref[...],
                                     preferred_element_type=jnp.float32)
                             + cb1_ref[...], 0.0)
            o_ref[...] = jnp.dot(zc, cw2_ref[...],
                                 preferred_element_type=jnp.float32) \
                + cb2_ref[...]

    return pl.pallas_call(
        body,
        grid=(_NB,),
        in_specs=[
            pl.BlockSpec((_BR, _H // 2), lambda i: (i, 0)),
            pl.BlockSpec((_BR, _H // 2), lambda i: (i, 0)),
            pl.BlockSpec((2, _BR, _H // 2), lambda i: (0, i, 0)),
            pl.BlockSpec((_H, _H), lambda i: (0, 0)),
            pl.BlockSpec((1, _H), lambda i: (0, 0)),
            pl.BlockSpec((_H, _H), lambda i: (0, 0)),
            pl.BlockSpec((1, _H), lambda i: (0, 0)),
            pl.BlockSpec((1, _H), lambda i: (0, 0)),
            pl.BlockSpec((1, _H), lambda i: (0, 0)),
            pl.BlockSpec((1, 1), lambda i: (0, 0)),
            pl.BlockSpec((1, _H), lambda i: (0, 0)),
            pl.BlockSpec((1, 1, _BR), lambda i: (i, 0, 0)),
            pl.BlockSpec((_H, _H), lambda i: (0, 0)),
            pl.BlockSpec((1, _H), lambda i: (0, 0)),
            pl.BlockSpec((_H, 3), lambda i: (0, 0)),
            pl.BlockSpec((1, 3), lambda i: (0, 0)),
        ],
        out_specs=pl.BlockSpec((_NG, 3), lambda i: (0, 0)),
        out_shape=jax.ShapeDtypeStruct((_NG, 3), jnp.float32),
        scratch_shapes=[
            pltpu.VMEM((_NG, _H), jnp.float32),
            pltpu.VMEM((_NG, 8), jnp.float32),
        ],
    )(xres[0], xres[1], agg, w1, b1.reshape(1, _H), w2, b2.reshape(1, _H),
      scale.reshape(1, _H), beta.reshape(1, _H), eps.reshape(1, 1),
      bprev.reshape(1, _H), batch3d, cw1, cb1.reshape(1, _H), cw2,
      cb2.reshape(1, 3))


_msg16 = _make_msg_kernel(16)
_msg32 = _make_msg_kernel(32)


def kernel(x, edge_index, batch, edge_attr, params):
    src2d = edge_index[0].reshape(_E // _CH, _CH)
    dst2d = edge_index[1].reshape(_E // _CH, _CH)
    a2d = edge_attr.reshape(_E)
    batch3d = batch.reshape(_NB, 1, _BR)

    lys = params["layers"]
    bn_eps = 1e-5

    be0 = lys[0]["edge_lin"]["b"]
    xt0 = jnp.pad(x[:, :2] + be0[:2], ((0, 0), (0, 14)))
    xt1 = jnp.pad(x[:, 2:4] + be0[2:], ((0, 0), (0, 14)))
    w0 = lys[0]["edge_lin"]["W"][0]
    wb0 = jnp.zeros((_NCORES, 2, 16), jnp.float32)
    wb0 = wb0.at[0, 0, :2].set(w0[:2]).at[1, 0, :2].set(w0[2:])
    agg0 = _msg16(xt0, xt1, src2d, dst2d, a2d, wb0)
    b1v = lys[1]["edge_lin"]["b"]
    b2v = lys[2]["edge_lin"]["b"]
    s0 = lys[0]["bn_gamma"] / jnp.sqrt(1.0 + bn_eps)
    tbl1 = _mlp_mid(x, agg0, lys[0]["nn1"]["W"], lys[0]["nn1"]["b"],
                    lys[0]["nn2"]["W"], lys[0]["nn2"]["b"], s0,
                    lys[0]["bn_beta"], lys[0]["eps"],
                    jnp.zeros((_H,), jnp.float32), b1v, 4, 16)

    w1v = lys[1]["edge_lin"]["W"][0]
    wb1 = jnp.stack([jnp.stack([w1v[:32], w1v[:32]]),
                     jnp.stack([w1v[32:], w1v[32:]])])
    agg1 = _msg32(tbl1[0], tbl1[1], src2d, dst2d, a2d, wb1)
    s1 = lys[1]["bn_gamma"] / jnp.sqrt(1.0 + bn_eps)
    tbl2 = _mlp_mid(tbl1, agg1, lys[1]["nn1"]["W"], lys[1]["nn1"]["b"],
                    lys[1]["nn2"]["W"], lys[1]["nn2"]["b"], s1,
                    lys[1]["bn_beta"], lys[1]["eps"], b1v, b2v, _H, 32)

    w2v = lys[2]["edge_lin"]["W"][0]
    wb2 = jnp.stack([jnp.stack([w2v[:32], w2v[:32]]),
                     jnp.stack([w2v[32:], w2v[32:]])])
    agg2 = _msg32(tbl2[0], tbl2[1], src2d, dst2d, a2d, wb2)
    s2 = lys[2]["bn_gamma"] / jnp.sqrt(1.0 + bn_eps)
    cls = params["cls"]
    return _mlp_pool(tbl2, agg2, lys[2]["nn1"]["W"], lys[2]["nn1"]["b"],
                     lys[2]["nn2"]["W"], lys[2]["nn2"]["b"], s2,
                     lys[2]["bn_beta"], lys[2]["eps"], b2v, batch3d,
                     cls["l1"]["W"], cls["l1"]["b"], cls["l2"]["W"],
                     cls["l2"]["b"])
